# baseline (speedup 1.0000x reference)
import functools

import jax
import jax.numpy as jnp
from jax import lax
from jax.experimental import pallas as pl
from jax.experimental.pallas import tpu as pltpu
from jax.experimental.pallas import tpu_sc as plsc

F = 26
V = 100000
D = 32
B = 16384
N = B * F
NC, NS, L = 2, 16, 16
NW = NC * NS
N_PER_W = N // NW
UNROLL = 8
VEC_ITERS = N_PER_W // (L * UNROLL)
CHUNK = 208
N_CHUNKS = N_PER_W // CHUNK
NBUF = 8
N_GROUPS = N_CHUNKS // NBUF - 1


def _sc_gather(x_flat, table_flat):
  mesh = plsc.VectorSubcoreMesh(core_axis_name="c", subcore_axis_name="s")

  @functools.partial(
      pl.kernel,
      out_type=jax.ShapeDtypeStruct((N, D), jnp.float32),
      mesh=mesh,
      scratch_types=[
          pltpu.VMEM((N_PER_W,), jnp.int32),
          pltpu.VMEM((NBUF, CHUNK, D), jnp.float32),
      ] + [pltpu.SemaphoreType.DMA] * NBUF,
      compiler_params=pltpu.CompilerParams(use_tc_tiling_on_sc=False),
  )
  def k(x_hbm, tab_hbm, out_hbm, idx_v, bufs, *sems):
    wid = lax.axis_index("s") * NC + lax.axis_index("c")
    base = wid * N_PER_W

    pltpu.sync_copy(x_hbm.at[pl.ds(base, N_PER_W)], idx_v)

    lanes = lax.iota(jnp.int32, L)

    def vbody(i, carry):
      for j in range(UNROLL):
        off = (i * UNROLL + j) * L
        fld = lax.rem(off + lanes, F)
        idx_v[pl.ds(off, L)] = idx_v[pl.ds(off, L)] + fld * V
      return carry

    lax.fori_loop(0, VEC_ITERS, vbody, 0)

    def gather(c, b):
      pltpu.async_copy(
          tab_hbm.at[idx_v.at[pl.ds(c * CHUNK, CHUNK)]], bufs.at[b], sems[b])

    def wait(b):
      pltpu.make_async_copy(
          tab_hbm.at[idx_v.at[pl.ds(0, CHUNK)]], bufs.at[b], sems[b]).wait()

    def writeout(c, b):
      pltpu.sync_copy(bufs.at[b], out_hbm.at[pl.ds(base + c * CHUNK, CHUNK)])

    for b in range(NBUF):
      gather(b, b)

    def gbody(g, carry):
      for b in range(NBUF):
        c = g * NBUF + b
        wait(b)
        writeout(c, b)
        gather(c + NBUF, b)
      return carry

    lax.fori_loop(0, N_GROUPS, gbody, 0)

    for b in range(NBUF):
      c = N_GROUPS * NBUF + b
      wait(b)
      writeout(c, b)

  return k(x_flat, table_flat)


def kernel(x_cat, tables):
  out = _sc_gather(x_cat.reshape(N), tables.reshape(F * V, D))
  return out.reshape(B, F, D)


# speedup vs baseline: 1.0007x; 1.0007x over previous
import functools

import jax
import jax.numpy as jnp
from jax import lax
from jax.experimental import pallas as pl
from jax.experimental.pallas import tpu as pltpu
from jax.experimental.pallas import tpu_sc as plsc

F = 26
V = 100000
D = 32
B = 16384
N = B * F
NC, NS, L = 2, 16, 16
NW = NC * NS
N_PER_W = N // NW
UNROLL = 8
VEC_ITERS = N_PER_W // (L * UNROLL)
CHUNK = 208
N_CHUNKS = N_PER_W // CHUNK
NBUF = 8
N_GROUPS = N_CHUNKS // NBUF - 1


def _sc_gather(x_flat, table_flat):
  mesh = plsc.VectorSubcoreMesh(core_axis_name="c", subcore_axis_name="s")

  @functools.partial(
      pl.kernel,
      out_type=jax.ShapeDtypeStruct((N, D), jnp.float32),
      mesh=mesh,
      scratch_types=[
          pltpu.VMEM((N_PER_W,), jnp.int32),
          pltpu.VMEM((NBUF, CHUNK, D), jnp.float32),
      ] + [pltpu.SemaphoreType.DMA] * NBUF,
      compiler_params=pltpu.CompilerParams(use_tc_tiling_on_sc=False),
  )
  def k(x_hbm, tab_hbm, out_hbm, idx_v, bufs, *sems):
    wid = lax.axis_index("s") * NC + lax.axis_index("c")
    base = wid * N_PER_W

    with jax.named_scope("stage_idx"):
      pltpu.sync_copy(x_hbm.at[pl.ds(base, N_PER_W)], idx_v)

    lanes = lax.iota(jnp.int32, L)

    def vbody(i, carry):
      for j in range(UNROLL):
        off = (i * UNROLL + j) * L
        fld = lax.rem(off + lanes, F)
        idx_v[pl.ds(off, L)] = idx_v[pl.ds(off, L)] + fld * V
      return carry

    with jax.named_scope("idx_arith"):
      lax.fori_loop(0, VEC_ITERS, vbody, 0)

    def gather(c, b):
      pltpu.async_copy(
          tab_hbm.at[idx_v.at[pl.ds(c * CHUNK, CHUNK)]], bufs.at[b], sems[b])

    def wait(b):
      pltpu.make_async_copy(
          tab_hbm.at[idx_v.at[pl.ds(0, CHUNK)]], bufs.at[b], sems[b]).wait()

    def writeout(c, b):
      pltpu.sync_copy(bufs.at[b], out_hbm.at[pl.ds(base + c * CHUNK, CHUNK)])

    with jax.named_scope("gather_pipe"):
      for b in range(NBUF):
        gather(b, b)

      def gbody(g, carry):
        for b in range(NBUF):
          c = g * NBUF + b
          wait(b)
          writeout(c, b)
          gather(c + NBUF, b)
        return carry

      lax.fori_loop(0, N_GROUPS, gbody, 0)

      for b in range(NBUF):
        c = N_GROUPS * NBUF + b
        wait(b)
        writeout(c, b)

  return k(x_flat, table_flat)


def kernel(x_cat, tables):
  out = _sc_gather(x_cat.reshape(N), tables.reshape(F * V, D))
  return out.reshape(B, F, D)
